# Initial kernel scaffold; baseline (speedup 1.0000x reference)
import jax
import jax.numpy as jnp
from jax.experimental import pallas as pl


def kernel(values_u, values_v, values_w):
    raise NotImplementedError("write your pallas kernel here")


# speedup vs baseline: 258.0526x; 258.0526x over previous
import jax
import jax.numpy as jnp
from jax.experimental import pallas as pl

NXK = 128
UBK = 1.0
BZ = 16


def _bc_kernel(u_ref, v_ref, w_ref, tu_ref, tv_ref, tw_ref):
    b = pl.program_id(0)
    u = u_ref[...]
    v = v_ref[...]
    w = w_ref[...]

    gz = jax.lax.broadcasted_iota(jnp.int32, (BZ, 1, 1), 0) + b * BZ
    y = jax.lax.broadcasted_iota(jnp.int32, (1, NXK, 1), 1)
    x = jax.lax.broadcasted_iota(jnp.int32, (1, 1, NXK), 2)

    out_u = jnp.where(y == 0, u[:, 1:2, :], jnp.where(y == NXK - 1, u[:, NXK - 2 : NXK - 1, :], u))
    x_edge = (x == 0) | (x == NXK - 1)
    y_int = (y >= 1) & (y <= NXK - 2)
    out_u = jnp.where(x_edge & y_int, jnp.float32(UBK), out_u)
    out_u = jnp.where(gz == 0, u[1:2, :, :], out_u)
    out_u = jnp.where(gz == NXK - 1, u[BZ - 2 : BZ - 1, :, :], out_u)

    bmask = (gz == 0) | (gz == NXK - 1) | (y == 0) | (y == NXK - 1) | x_edge
    zero = jnp.float32(0.0)
    tu_ref[...] = out_u
    tv_ref[...] = jnp.where(bmask, zero, v)
    tw_ref[...] = jnp.where(bmask, zero, w)


def kernel(values_u, values_v, values_w):
    u = values_u.reshape(NXK, NXK, NXK)
    v = values_v.reshape(NXK, NXK, NXK)
    w = values_w.reshape(NXK, NXK, NXK)
    spec = pl.BlockSpec((BZ, NXK, NXK), lambda i: (i, 0, 0))
    out = pl.pallas_call(
        _bc_kernel,
        grid=(NXK // BZ,),
        in_specs=[spec, spec, spec],
        out_specs=[spec, spec, spec],
        out_shape=[jax.ShapeDtypeStruct((NXK, NXK, NXK), jnp.float32)] * 3,
    )(u, v, w)
    shp = values_u.shape
    return (out[0].reshape(shp), out[1].reshape(shp), out[2].reshape(shp))
